# baseline (speedup 1.0000x reference)
import functools

import jax
import jax.numpy as jnp
from jax import lax
from jax.experimental import pallas as pl
from jax.experimental.pallas import tpu as pltpu
from jax.experimental.pallas import tpu_sc as plsc

NC = 2
NS = 16
LANES = 16

NP = 122880
BK1, NB1 = 8192, 15
BK2, NB2 = 8192, 15
CH = 2048
ROWB = 64
RING = 4

MBLK = 512


def _segsum_sc(width, bk, nb, np_, e_pad):
    ept = e_pad // NS
    nch = ept // CH
    accr = bk + 512
    zpt = accr // NS
    wpt = bk // NS
    nbc = (nb + 1) // NC
    shf = ROWB.bit_length() - 1
    zr = 16
    cflat = CH + ROWB
    mesh = plsc.VectorSubcoreMesh(core_axis_name="c", subcore_axis_name="s")

    @functools.partial(
        pl.kernel,
        out_type=jax.ShapeDtypeStruct((np_, width), jnp.float32),
        mesh=mesh,
        compiler_params=pltpu.CompilerParams(
            needs_layout_passes=False, use_tc_tiling_on_sc=False),
        scratch_types=[
            pltpu.VMEM_SHARED((accr, width), jnp.float32),
            pltpu.VMEM((2, CH), jnp.int32),
            pltpu.VMEM((2, CH), jnp.int32),
            pltpu.VMEM((cflat,), jnp.int32),
            pltpu.VMEM((cflat,), jnp.int32),
            pltpu.VMEM((RING, ROWB, width), jnp.float32),
            pltpu.VMEM((zr, width), jnp.float32),
            pltpu.SemaphoreType.DMA,
            pltpu.SemaphoreType.DMA,
            pltpu.SemaphoreType.DMA,
            pltpu.SemaphoreType.DMA,
            pltpu.SemaphoreType.DMA,
            pltpu.SemaphoreType.DMA,
            pltpu.SemaphoreType.DMA,
            pltpu.SemaphoreType.DMA,
            pltpu.SemaphoreType.DMA,
            pltpu.SemaphoreType.DMA,
            pltpu.SemaphoreType.DMA,
        ],
    )
    def seg(x_hbm, src_hbm, dst_hbm, out_hbm,
            acc, dstb, srcb, csrc, cdst, rows, zbuf,
            esem0, esem1, g0, g1, g2, g3, s0, s1, s2, s3, zsem):
        c = lax.axis_index("c")
        s = lax.axis_index("s")
        zero_f = jnp.zeros((LANES,), jnp.float32)
        zero_i = jnp.zeros((LANES,), jnp.int32)
        iota = lax.iota(jnp.int32, LANES)
        dummy = bk + s * 8
        ebase = s * ept
        esems = (esem0, esem1)
        gsems = (g0, g1, g2, g3)
        ssems = (s0, s1, s2, s3)

        def zb_body(r, carry):
            for k in range(width // LANES):
                zbuf[r, pl.ds(k * LANES, LANES)] = zero_f
            return carry
        lax.fori_loop(0, zr, zb_body, 0)

        def start_edges(chi, eb):
            off = ebase + chi * CH
            pltpu.async_copy(dst_hbm.at[pl.ds(off, CH)], dstb.at[eb],
                             esems[eb])
            pltpu.async_copy(src_hbm.at[pl.ds(off, CH)], srcb.at[eb],
                             esems[eb])

        def wait_edges(chi, eb):
            off = ebase + chi * CH
            pltpu.make_async_copy(dst_hbm.at[pl.ds(off, CH)], dstb.at[eb],
                                  esems[eb]).wait()
            pltpu.make_async_copy(src_hbm.at[pl.ds(off, CH)], srcb.at[eb],
                                  esems[eb]).wait()

        def start_gather(g, r):
            pltpu.async_copy(x_hbm.at[csrc.at[pl.ds(g * ROWB, ROWB)]],
                             rows.at[r], gsems[r])

        def wait_gather(g, r):
            pltpu.make_async_copy(x_hbm.at[csrc.at[pl.ds(g * ROWB, ROWB)]],
                                  rows.at[r], gsems[r]).wait()

        def start_scatter(g, r):
            pltpu.async_copy(rows.at[r],
                             acc.at[cdst.at[pl.ds(g * ROWB, ROWB)]],
                             ssems[r], add=True)

        def wait_scatter(g, r):
            pltpu.make_async_copy(rows.at[r],
                                  acc.at[cdst.at[pl.ds(g * ROWB, ROWB)]],
                                  ssems[r]).wait()

        def pass_body(p, carry):
            b = c * nbc + p
            lo = b * bk

            @pl.when(b < nb)
            def _do_pass():
                def z_start(i, cz):
                    pltpu.async_copy(
                        zbuf, acc.at[pl.ds(s * zpt + i * zr, zr)], zsem)
                    return cz
                lax.fori_loop(0, zpt // zr, z_start, 0)

                def z_drain(i, cz):
                    pltpu.make_async_copy(
                        zbuf, acc.at[pl.ds(s * zpt + i * zr, zr)],
                        zsem).wait()
                    return cz
                lax.fori_loop(0, zpt // zr, z_drain, 0)
                plsc.subcore_barrier()

                start_edges(0, 0)

                def chunk_body(chi, cc):
                    eb = chi & 1

                    @pl.when(eb == 0)
                    def _w0():
                        wait_edges(chi, 0)

                        @pl.when(chi + 1 < nch)
                        def _p1():
                            start_edges(chi + 1, 1)

                    @pl.when(eb == 1)
                    def _w1():
                        wait_edges(chi, 1)

                        @pl.when(chi + 1 < nch)
                        def _p0():
                            start_edges(chi + 1, 0)

                    @plsc.parallel_loop(0, CH // LANES, 1, unroll=8,
                                        carry=jnp.int32(0))
                    def k_in(j, kacc):
                        dv = dstb[eb, pl.ds(j * LANES, LANES)]
                        sv = srcb[eb, pl.ds(j * LANES, LANES)]
                        dvl = dv - lo
                        m = (dvl >= 0) & (dvl < bk)
                        cnt = plsc.all_reduce_population_count(m)
                        plsc.store_compressed(csrc.at[pl.ds(kacc, LANES)],
                                              sv, mask=m)
                        plsc.store_compressed(cdst.at[pl.ds(kacc, LANES)],
                                              dvl, mask=m)
                        return kacc + cnt[0]

                    ng = (k_in + (ROWB - 1)) >> shf
                    kpad = ng << shf
                    for t in range(ROWB // LANES):
                        pos = k_in + t * LANES + iota
                        mm = pos < kpad
                        plsc.store_scatter(csrc, [pos], zero_i, mask=mm)
                        plsc.store_scatter(cdst, [pos], zero_i + dummy,
                                           mask=mm)

                    for gg in range(RING - 1):
                        @pl.when(gg < ng)
                        def _pro(gg=gg):
                            start_gather(gg, gg)

                    def gs_body(g, cg):
                        sl = g & (RING - 1)
                        for r in range(RING):
                            @pl.when(sl == r)
                            def _gs(r=r):
                                wait_gather(g, r)
                                start_scatter(g, r)

                        @pl.when(g + RING - 1 < ng)
                        def _next():
                            sl2 = (g + RING - 1) & (RING - 1)
                            for r in range(RING):
                                @pl.when(sl2 == r)
                                def _n(r=r):
                                    @pl.when(g >= 1)
                                    def _ws():
                                        wait_scatter(g - 1, r)
                                    start_gather(g + RING - 1, r)
                        return cg
                    lax.fori_loop(0, ng, gs_body, 0)

                    for t in range(RING):
                        @pl.when(ng - 1 - t >= 0)
                        def _dr(t=t):
                            gd = ng - 1 - t
                            sl = gd & (RING - 1)
                            for r in range(RING):
                                @pl.when(sl == r)
                                def _d(r=r):
                                    wait_scatter(gd, r)
                    return cc
                lax.fori_loop(0, nch, chunk_body, 0)
                plsc.subcore_barrier()

                pltpu.sync_copy(acc.at[pl.ds(s * wpt, wpt)],
                                out_hbm.at[pl.ds(lo + s * wpt, wpt)])
                plsc.subcore_barrier()
            return carry

        lax.fori_loop(0, nbc, pass_body, 0)

    return seg


def _fuse_mm(x_ref, w_ref, o_ref):
    o_ref[...] = jnp.dot(x_ref[...], w_ref[...],
                         preferred_element_type=jnp.float32)


def _layer1_body(s_ref, cnt_ref, x0_ref, wl1_ref, b1_ref, wr1_ref,
                 wl2_ref, wr2_ref, b2_ref, y1_ref, r1b_ref):
    inv = 1.0 / jnp.maximum(cnt_ref[:, 128:129], 1.0)
    agg = s_ref[...] * inv
    x1 = (jnp.dot(agg, wl1_ref[...], preferred_element_type=jnp.float32)
          + b1_ref[...]
          + jnp.dot(x0_ref[...], wr1_ref[...],
                    preferred_element_type=jnp.float32))
    x1 = jnp.where(x1 > 0, x1, 0.01 * x1)
    y1_ref[...] = jnp.dot(x1, wl2_ref[...], preferred_element_type=jnp.float32)
    r1b_ref[...] = (jnp.dot(x1, wr2_ref[...],
                            preferred_element_type=jnp.float32) + b2_ref[...])


def _layer2_body(s1_ref, cnt_ref, r1b_ref, o_ref):
    inv = 1.0 / jnp.maximum(cnt_ref[:, 128:129], 1.0)
    o_ref[...] = s1_ref[...] * inv + r1b_ref[...]


def kernel(feature, edge_index, user, W_fuse, W_l1, b_l1, W_r1,
           W_l2, b_l2, W_r2):
    n_user = user.shape[0]
    n_item = feature.shape[0]
    n = n_user + n_item
    e = edge_index.shape[1]

    e_pad = -(-e // (NS * CH)) * (NS * CH)
    src = jnp.pad(edge_index[0], (0, e_pad - e))
    dst = jnp.pad(edge_index[1], (0, e_pad - e), constant_values=NP)

    item_pad = -(-n_item // MBLK) * MBLK
    feat_p = jnp.pad(feature, ((0, item_pad - n_item), (0, 0)))
    item_x = pl.pallas_call(
        _fuse_mm,
        grid=(item_pad // MBLK,),
        in_specs=[pl.BlockSpec((MBLK, 384), lambda i: (i, 0)),
                  pl.BlockSpec((384, 128), lambda i: (0, 0))],
        out_specs=pl.BlockSpec((MBLK, 128), lambda i: (i, 0)),
        out_shape=jax.ShapeDtypeStruct((item_pad, 128), jnp.float32),
    )(feat_p, W_fuse.T)

    x0 = jnp.concatenate([user, item_x[:n_item]], axis=0)
    x0 = jnp.pad(x0, ((0, NP - n), (0, 0)))
    x0ext = jnp.concatenate(
        [x0, jnp.ones((NP, 1), jnp.float32), jnp.zeros((NP, 15), jnp.float32)],
        axis=1)

    s0ext = _segsum_sc(144, BK1, NB1, NP, e_pad)(x0ext, src, dst)

    y1, r1b = pl.pallas_call(
        _layer1_body,
        grid=(NP // MBLK,),
        in_specs=[pl.BlockSpec((MBLK, 128), lambda i: (i, 0)),
                  pl.BlockSpec((MBLK, 144), lambda i: (i, 0)),
                  pl.BlockSpec((MBLK, 128), lambda i: (i, 0)),
                  pl.BlockSpec((128, 128), lambda i: (0, 0)),
                  pl.BlockSpec((1, 128), lambda i: (0, 0)),
                  pl.BlockSpec((128, 128), lambda i: (0, 0)),
                  pl.BlockSpec((128, 64), lambda i: (0, 0)),
                  pl.BlockSpec((128, 64), lambda i: (0, 0)),
                  pl.BlockSpec((1, 64), lambda i: (0, 0))],
        out_specs=[pl.BlockSpec((MBLK, 64), lambda i: (i, 0)),
                   pl.BlockSpec((MBLK, 64), lambda i: (i, 0))],
        out_shape=[jax.ShapeDtypeStruct((NP, 64), jnp.float32),
                   jax.ShapeDtypeStruct((NP, 64), jnp.float32)],
    )(s0ext, s0ext, x0ext, W_l1.T, b_l1.reshape(1, 128), W_r1.T,
      W_l2.T, W_r2.T, b_l2.reshape(1, 64))

    s1 = _segsum_sc(64, BK2, NB2, NP, e_pad)(y1, src, dst)

    x2 = pl.pallas_call(
        _layer2_body,
        grid=(NP // MBLK,),
        in_specs=[pl.BlockSpec((MBLK, 64), lambda i: (i, 0)),
                  pl.BlockSpec((MBLK, 144), lambda i: (i, 0)),
                  pl.BlockSpec((MBLK, 64), lambda i: (i, 0))],
        out_specs=pl.BlockSpec((MBLK, 64), lambda i: (i, 0)),
        out_shape=jax.ShapeDtypeStruct((NP, 64), jnp.float32),
    )(s1, s0ext, r1b)

    return x2[:n]


# speedup vs baseline: 1.8553x; 1.8553x over previous
import functools

import jax
import jax.numpy as jnp
from jax import lax
from jax.experimental import pallas as pl
from jax.experimental.pallas import tpu as pltpu
from jax.experimental.pallas import tpu_sc as plsc

NC = 2
NS = 16
LANES = 16

NP = 122880
BK1, NB1 = 10240, 12
BK2, NB2 = 20480, 6
CH = 2048
ROWB = 64
RING = 4

MBLK = 512


def _segsum_sc(width, bk, nb, np_, e_pad):
    ept = e_pad // NS
    nch = ept // CH
    accr = bk + 128
    zpt = accr // NS
    wpt = bk // NS
    nbc = (nb + 1) // NC
    shf = ROWB.bit_length() - 1
    zr = 8
    cflat = CH + ROWB
    mesh = plsc.VectorSubcoreMesh(core_axis_name="c", subcore_axis_name="s")

    @functools.partial(
        pl.kernel,
        out_type=jax.ShapeDtypeStruct((np_, width), jnp.float32),
        mesh=mesh,
        compiler_params=pltpu.CompilerParams(
            needs_layout_passes=False, use_tc_tiling_on_sc=False),
        scratch_types=[
            pltpu.VMEM_SHARED((accr, width), jnp.float32),
            pltpu.VMEM((2, CH), jnp.int32),
            pltpu.VMEM((2, CH), jnp.int32),
            pltpu.VMEM((cflat,), jnp.int32),
            pltpu.VMEM((cflat,), jnp.int32),
            pltpu.VMEM((RING, ROWB, width), jnp.float32),
            pltpu.VMEM((zr, width), jnp.float32),
            pltpu.SemaphoreType.DMA,
            pltpu.SemaphoreType.DMA,
            pltpu.SemaphoreType.DMA,
            pltpu.SemaphoreType.DMA,
            pltpu.SemaphoreType.DMA,
            pltpu.SemaphoreType.DMA,
            pltpu.SemaphoreType.DMA,
            pltpu.SemaphoreType.DMA,
            pltpu.SemaphoreType.DMA,
            pltpu.SemaphoreType.DMA,
            pltpu.SemaphoreType.DMA,
        ],
    )
    def seg(x_hbm, src_hbm, dst_hbm, out_hbm,
            acc, dstb, srcb, csrc, cdst, rows, zbuf,
            esem0, esem1, g0, g1, g2, g3, s0, s1, s2, s3, zsem):
        c = lax.axis_index("c")
        s = lax.axis_index("s")
        zero_f = jnp.zeros((LANES,), jnp.float32)
        zero_i = jnp.zeros((LANES,), jnp.int32)
        iota = lax.iota(jnp.int32, LANES)
        dummy = bk + s * 8
        ebase = s * ept
        esems = (esem0, esem1)
        gsems = (g0, g1, g2, g3)
        ssems = (s0, s1, s2, s3)

        def zb_body(r, carry):
            for k in range(width // LANES):
                zbuf[r, pl.ds(k * LANES, LANES)] = zero_f
            return carry
        lax.fori_loop(0, zr, zb_body, 0)

        def start_edges(chi, eb):
            off = ebase + chi * CH
            pltpu.async_copy(dst_hbm.at[pl.ds(off, CH)], dstb.at[eb],
                             esems[eb])
            pltpu.async_copy(src_hbm.at[pl.ds(off, CH)], srcb.at[eb],
                             esems[eb])

        def wait_edges(chi, eb):
            off = ebase + chi * CH
            pltpu.make_async_copy(dst_hbm.at[pl.ds(off, CH)], dstb.at[eb],
                                  esems[eb]).wait()
            pltpu.make_async_copy(src_hbm.at[pl.ds(off, CH)], srcb.at[eb],
                                  esems[eb]).wait()

        def start_gather(g, r):
            pltpu.async_copy(x_hbm.at[csrc.at[pl.ds(g * ROWB, ROWB)]],
                             rows.at[r], gsems[r])

        def wait_gather(g, r):
            pltpu.make_async_copy(x_hbm.at[csrc.at[pl.ds(g * ROWB, ROWB)]],
                                  rows.at[r], gsems[r]).wait()

        def start_scatter(g, r):
            pltpu.async_copy(rows.at[r],
                             acc.at[cdst.at[pl.ds(g * ROWB, ROWB)]],
                             ssems[r], add=True)

        def wait_scatter(g, r):
            pltpu.make_async_copy(rows.at[r],
                                  acc.at[cdst.at[pl.ds(g * ROWB, ROWB)]],
                                  ssems[r]).wait()

        def pass_body(p, carry):
            b = c * nbc + p
            lo = b * bk

            @pl.when(b < nb)
            def _do_pass():
                def z_start(i, cz):
                    pltpu.async_copy(
                        zbuf, acc.at[pl.ds(s * zpt + i * zr, zr)], zsem)
                    return cz
                lax.fori_loop(0, zpt // zr, z_start, 0)

                def z_drain(i, cz):
                    pltpu.make_async_copy(
                        zbuf, acc.at[pl.ds(s * zpt + i * zr, zr)],
                        zsem).wait()
                    return cz
                lax.fori_loop(0, zpt // zr, z_drain, 0)
                plsc.subcore_barrier()

                start_edges(0, 0)

                def chunk_body(chi, cc):
                    eb = chi & 1

                    @pl.when(eb == 0)
                    def _w0():
                        wait_edges(chi, 0)

                        @pl.when(chi + 1 < nch)
                        def _p1():
                            start_edges(chi + 1, 1)

                    @pl.when(eb == 1)
                    def _w1():
                        wait_edges(chi, 1)

                        @pl.when(chi + 1 < nch)
                        def _p0():
                            start_edges(chi + 1, 0)

                    @plsc.parallel_loop(0, CH // LANES, 1, unroll=8,
                                        carry=jnp.int32(0))
                    def k_in(j, kacc):
                        dv = dstb[eb, pl.ds(j * LANES, LANES)]
                        sv = srcb[eb, pl.ds(j * LANES, LANES)]
                        dvl = dv - lo
                        m = (dvl >= 0) & (dvl < bk)
                        cnt = plsc.all_reduce_population_count(m)
                        nhit = cnt[0]

                        @pl.when(nhit > 0)
                        def _store():
                            plsc.store_compressed(
                                csrc.at[pl.ds(kacc, LANES)], sv, mask=m)
                            plsc.store_compressed(
                                cdst.at[pl.ds(kacc, LANES)], dvl, mask=m)
                        return kacc + nhit

                    ng = (k_in + (ROWB - 1)) >> shf
                    kpad = ng << shf
                    for t in range(ROWB // LANES):
                        pos = k_in + t * LANES + iota
                        mm = pos < kpad
                        plsc.store_scatter(csrc, [pos], zero_i, mask=mm)
                        plsc.store_scatter(cdst, [pos], zero_i + dummy,
                                           mask=mm)

                    for gg in range(RING - 1):
                        @pl.when(gg < ng)
                        def _pro(gg=gg):
                            start_gather(gg, gg)

                    def gs_body(g, cg):
                        sl = g & (RING - 1)
                        for r in range(RING):
                            @pl.when(sl == r)
                            def _gs(r=r):
                                wait_gather(g, r)
                                start_scatter(g, r)

                        @pl.when(g + RING - 1 < ng)
                        def _next():
                            sl2 = (g + RING - 1) & (RING - 1)
                            for r in range(RING):
                                @pl.when(sl2 == r)
                                def _n(r=r):
                                    @pl.when(g >= 1)
                                    def _ws():
                                        wait_scatter(g - 1, r)
                                    start_gather(g + RING - 1, r)
                        return cg
                    lax.fori_loop(0, ng, gs_body, 0)

                    for t in range(RING):
                        @pl.when(ng - 1 - t >= 0)
                        def _dr(t=t):
                            gd = ng - 1 - t
                            sl = gd & (RING - 1)
                            for r in range(RING):
                                @pl.when(sl == r)
                                def _d(r=r):
                                    wait_scatter(gd, r)
                    return cc
                lax.fori_loop(0, nch, chunk_body, 0)
                plsc.subcore_barrier()

                pltpu.sync_copy(acc.at[pl.ds(s * wpt, wpt)],
                                out_hbm.at[pl.ds(lo + s * wpt, wpt)])
                plsc.subcore_barrier()
            return carry

        lax.fori_loop(0, nbc, pass_body, 0)

    return seg


def _degree_sc(np_, e_pad):
    npp = np_ + 512
    ept = e_pad // NS
    nch = ept // CH
    mesh = plsc.VectorSubcoreMesh(core_axis_name="c", subcore_axis_name="s")

    @functools.partial(
        pl.kernel,
        out_type=jax.ShapeDtypeStruct((NC * NS, npp), jnp.float32),
        mesh=mesh,
        compiler_params=pltpu.CompilerParams(
            needs_layout_passes=False, use_tc_tiling_on_sc=False),
        scratch_types=[
            pltpu.VMEM((npp,), jnp.float32),
            pltpu.VMEM((2, CH), jnp.int32),
            pltpu.SemaphoreType.DMA,
            pltpu.SemaphoreType.DMA,
        ],
    )
    def deg(dst_hbm, out_hbm, histo, dstb, esem0, esem1):
        c = lax.axis_index("c")
        s = lax.axis_index("s")
        zero_f = jnp.zeros((LANES,), jnp.float32)
        ones_f = jnp.full((LANES,), 1.0, jnp.float32)
        ebase = s * ept
        esems = (esem0, esem1)

        @plsc.parallel_loop(0, npp // LANES, 1, unroll=8)
        def _zero(i):
            histo[pl.ds(i * LANES, LANES)] = zero_f

        def start_edges(chi, eb):
            off = ebase + chi * CH
            pltpu.async_copy(dst_hbm.at[pl.ds(off, CH)], dstb.at[eb],
                             esems[eb])

        def wait_edges(chi, eb):
            off = ebase + chi * CH
            pltpu.make_async_copy(dst_hbm.at[pl.ds(off, CH)], dstb.at[eb],
                                  esems[eb]).wait()

        start_edges(0, 0)

        def chunk_body(chi, cc):
            eb = chi & 1

            @pl.when(eb == 0)
            def _w0():
                wait_edges(chi, 0)

                @pl.when(chi + 1 < nch)
                def _p1():
                    start_edges(chi + 1, 1)

            @pl.when(eb == 1)
            def _w1():
                wait_edges(chi, 1)

                @pl.when(chi + 1 < nch)
                def _p0():
                    start_edges(chi + 1, 0)

            def h_body(j, ch):
                dv = dstb[eb, pl.ds(j * LANES, LANES)]
                plsc.addupdate_scatter(histo, [dv], ones_f)
                return ch
            lax.fori_loop(0, CH // LANES, h_body, 0)
            return cc
        lax.fori_loop(0, nch, chunk_body, 0)

        pltpu.sync_copy(histo, out_hbm.at[c * NS + s])

    return deg


def _fuse_mm(x_ref, w_ref, o_ref):
    o_ref[...] = jnp.dot(x_ref[...], w_ref[...],
                         preferred_element_type=jnp.float32)


def _layer1_body(s_ref, deg_ref, x0_ref, wl1_ref, b1_ref, wr1_ref,
                 wl2_ref, wr2_ref, b2_ref, y1_ref, r1b_ref):
    deg2 = jnp.sum(deg_ref[...], axis=0)
    inv = (2.0 / jnp.maximum(deg2, 2.0))[:, None]
    agg = s_ref[...] * inv
    x1 = (jnp.dot(agg, wl1_ref[...], preferred_element_type=jnp.float32)
          + b1_ref[...]
          + jnp.dot(x0_ref[...], wr1_ref[...],
                    preferred_element_type=jnp.float32))
    x1 = jnp.where(x1 > 0, x1, 0.01 * x1)
    y1_ref[...] = jnp.dot(x1, wl2_ref[...], preferred_element_type=jnp.float32)
    r1b_ref[...] = (jnp.dot(x1, wr2_ref[...],
                            preferred_element_type=jnp.float32) + b2_ref[...])


def _layer2_body(s1_ref, deg_ref, r1b_ref, o_ref):
    deg2 = jnp.sum(deg_ref[...], axis=0)
    inv = (2.0 / jnp.maximum(deg2, 2.0))[:, None]
    o_ref[...] = s1_ref[...] * inv + r1b_ref[...]


def kernel(feature, edge_index, user, W_fuse, W_l1, b_l1, W_r1,
           W_l2, b_l2, W_r2):
    n_user = user.shape[0]
    n_item = feature.shape[0]
    n = n_user + n_item
    e = edge_index.shape[1]

    e_pad = -(-e // (NS * CH)) * (NS * CH)
    src = jnp.pad(edge_index[0], (0, e_pad - e))
    dst = jnp.pad(edge_index[1], (0, e_pad - e), constant_values=NP)

    item_pad = -(-n_item // MBLK) * MBLK
    feat_p = jnp.pad(feature, ((0, item_pad - n_item), (0, 0)))
    item_x = pl.pallas_call(
        _fuse_mm,
        grid=(item_pad // MBLK,),
        in_specs=[pl.BlockSpec((MBLK, 384), lambda i: (i, 0)),
                  pl.BlockSpec((384, 128), lambda i: (0, 0))],
        out_specs=pl.BlockSpec((MBLK, 128), lambda i: (i, 0)),
        out_shape=jax.ShapeDtypeStruct((item_pad, 128), jnp.float32),
    )(feat_p, W_fuse.T)

    x0 = jnp.concatenate([user, item_x[:n_item]], axis=0)
    x0 = jnp.pad(x0, ((0, NP - n), (0, 0)))

    degp = _degree_sc(NP, e_pad)(dst)
    s0 = _segsum_sc(128, BK1, NB1, NP, e_pad)(x0, src, dst)

    y1, r1b = pl.pallas_call(
        _layer1_body,
        grid=(NP // MBLK,),
        in_specs=[pl.BlockSpec((MBLK, 128), lambda i: (i, 0)),
                  pl.BlockSpec((32, MBLK), lambda i: (0, i)),
                  pl.BlockSpec((MBLK, 128), lambda i: (i, 0)),
                  pl.BlockSpec((128, 128), lambda i: (0, 0)),
                  pl.BlockSpec((1, 128), lambda i: (0, 0)),
                  pl.BlockSpec((128, 128), lambda i: (0, 0)),
                  pl.BlockSpec((128, 64), lambda i: (0, 0)),
                  pl.BlockSpec((128, 64), lambda i: (0, 0)),
                  pl.BlockSpec((1, 64), lambda i: (0, 0))],
        out_specs=[pl.BlockSpec((MBLK, 64), lambda i: (i, 0)),
                   pl.BlockSpec((MBLK, 64), lambda i: (i, 0))],
        out_shape=[jax.ShapeDtypeStruct((NP, 64), jnp.float32),
                   jax.ShapeDtypeStruct((NP, 64), jnp.float32)],
    )(s0, degp, x0, W_l1.T, b_l1.reshape(1, 128), W_r1.T,
      W_l2.T, W_r2.T, b_l2.reshape(1, 64))

    s1 = _segsum_sc(64, BK2, NB2, NP, e_pad)(y1, src, dst)

    x2 = pl.pallas_call(
        _layer2_body,
        grid=(NP // MBLK,),
        in_specs=[pl.BlockSpec((MBLK, 64), lambda i: (i, 0)),
                  pl.BlockSpec((32, MBLK), lambda i: (0, i)),
                  pl.BlockSpec((MBLK, 64), lambda i: (i, 0))],
        out_specs=pl.BlockSpec((MBLK, 64), lambda i: (i, 0)),
        out_shape=jax.ShapeDtypeStruct((NP, 64), jnp.float32),
    )(s1, degp, r1b)

    return x2[:n]
